# baseline (speedup 1.0000x reference)
import functools

import jax
import jax.numpy as jnp
from jax import lax
from jax.experimental import pallas as pl
from jax.experimental.pallas import tpu as pltpu
from jax.experimental.pallas import tpu_sc as plsc

NACC = 10240
LCH = 128
NT = 32

_SC_PARAMS = pltpu.CompilerParams(needs_layout_passes=False)


def _make_deg_kernel(tc):
    mesh = plsc.VectorSubcoreMesh(core_axis_name="c", subcore_axis_name="s")

    @functools.partial(
        pl.kernel, mesh=mesh,
        out_type=jax.ShapeDtypeStruct((NT, 4, NACC), jnp.float32),
        compiler_params=_SC_PARAMS,
        scratch_types=[
            pltpu.VMEM((tc,), jnp.int32),
            pltpu.VMEM((tc,), jnp.int32),
            pltpu.VMEM((tc,), jnp.int32),
            pltpu.VMEM((tc,), jnp.int32),
            pltpu.VMEM((NACC,), jnp.float32),
            pltpu.VMEM((NACC,), jnp.float32),
            pltpu.VMEM((NACC,), jnp.float32),
            pltpu.VMEM((NACC,), jnp.float32),
            pltpu.SemaphoreType.DMA,
        ],
    )
    def deg_kernel(idx_hbm, out_hbm, i0, i1, i2, i3,
                   a0, a1, a2, a3, sem):
        cid = lax.axis_index("c")
        sid = lax.axis_index("s")
        wid = sid * 2 + cid
        accs = [a0, a1, a2, a3]
        idxs = [i0, i1, i2, i3]
        ones = jnp.ones((16,), jnp.float32)
        zero = jnp.zeros((16,), jnp.float32)

        def zb(j, c):
            for a in range(4):
                accs[a][pl.ds(j * 16, 16)] = zero
            return c
        lax.fori_loop(0, NACC // 16, zb, 0)
        descs = [pltpu.async_copy(idx_hbm.at[a, wid], idxs[a], sem)
                 for a in range(4)]
        for d in descs:
            d.wait()

        def sb(j, c):
            for u in range(2):
                for a in range(4):
                    v = idxs[a][pl.ds(j * 32 + u * 16, 16)]
                    plsc.addupdate_scatter(accs[a], [v], ones)
            return c
        lax.fori_loop(0, tc // 32, sb, 0)
        plsc.subcore_barrier()
        for a in range(4):
            pltpu.sync_copy(accs[a], out_hbm.at[wid, a])

    return deg_kernel


def _make_edge_kernel(tc):
    mesh = plsc.VectorSubcoreMesh(core_axis_name="c", subcore_axis_name="s")

    @functools.partial(
        pl.kernel, mesh=mesh,
        out_type=jax.ShapeDtypeStruct((NT, 2, NACC), jnp.float32),
        compiler_params=_SC_PARAMS,
        scratch_types=[
            pltpu.VMEM((tc,), jnp.int32),
            pltpu.VMEM((tc,), jnp.int32),
            pltpu.VMEM((tc,), jnp.int32),
            pltpu.VMEM((tc,), jnp.int32),
            pltpu.VMEM((NACC,), jnp.float32),
            pltpu.VMEM((NACC,), jnp.float32),
            pltpu.VMEM((NACC,), jnp.float32),
            pltpu.VMEM((NACC,), jnp.float32),
            pltpu.SemaphoreType.DMA,
        ],
    )
    def edge_kernel(q_hbm, idx_hbm, out_hbm, s0, s1, d0, d1,
                    q0, q1, a0, a1, sem):
        cid = lax.axis_index("c")
        sid = lax.axis_index("s")
        wid = sid * 2 + cid
        sidx = [s0, s1]
        didx = [d0, d1]
        qv = [q0, q1]
        accs = [a0, a1]
        zero = jnp.zeros((16,), jnp.float32)

        def zb(j, c):
            for s in range(2):
                accs[s][pl.ds(j * 16, 16)] = zero
            return c
        lax.fori_loop(0, NACC // 16, zb, 0)
        descs = []
        for s in range(2):
            descs.append(pltpu.async_copy(idx_hbm.at[2 * s, wid], sidx[s], sem))
            descs.append(pltpu.async_copy(idx_hbm.at[2 * s + 1, wid], didx[s], sem))
            descs.append(pltpu.async_copy(q_hbm.at[s], qv[s], sem))
        for d in descs:
            d.wait()

        def eb(j, c):
            for u in range(2):
                for s in range(2):
                    sv = sidx[s][pl.ds(j * 32 + u * 16, 16)]
                    vals = plsc.load_gather(qv[s], [sv])
                    dv = didx[s][pl.ds(j * 32 + u * 16, 16)]
                    plsc.addupdate_scatter(accs[s], [dv], vals)
            return c
        lax.fori_loop(0, tc // 32, eb, 0)
        plsc.subcore_barrier()
        for s in range(2):
            pltpu.sync_copy(accs[s], out_hbm.at[wid, s])

    return edge_kernel


def _mv_kernel(dx_ref, px_ref, wdp_ref, w1_ref, wpd_ref, w2_ref, s_ref):
    u1 = jnp.dot(wdp_ref[...], w1_ref[...], preferred_element_type=jnp.float32)
    u2 = jnp.dot(wpd_ref[...], w2_ref[...], preferred_element_type=jnp.float32)
    s_ref[0, :] = jnp.dot(dx_ref[...], u1, preferred_element_type=jnp.float32)[:, 0]
    s_ref[1, :] = jnp.dot(px_ref[...], u2, preferred_element_type=jnp.float32)[:, 0]


def _q_kernel(s_ref, deg_ref, q_ref):
    deg = jnp.sum(deg_ref[...], axis=0)
    r = lax.rsqrt(jnp.maximum(deg, 1.0))
    q_ref[0, :] = s_ref[0] * r[0]
    q_ref[1, :] = s_ref[1] * r[2]
    q_ref[2, :] = r[1]
    q_ref[3, :] = r[3]


def _fin_kernel(t_ref, q_ref, bdp_ref, bpd_ref, w1_ref, w2_ref, bl_ref, o_ref):
    c1 = (jnp.sum(bdp_ref[...] * w1_ref[...])
          + jnp.sum(bpd_ref[...] * w2_ref[...]) + bl_ref[0, 0])
    t = jnp.sum(t_ref[...], axis=0)
    z = t[0] * q_ref[2, :] + t[1] * q_ref[3, :] + c1
    o_ref[...] = 1.0 / (1.0 + jnp.exp(-z))


def kernel(drug_x, protein_x, edge_index, rev_edge_index, W_drug_lin,
           b_drug_lin, W_protein_lin, b_protein_lin, conv_W_dp, conv_b_dp,
           conv_W_pd, conv_b_pd, W_link, b_link):
    n = drug_x.shape[0]
    d_h = conv_W_dp.shape[2]
    e = edge_index.shape[1]
    tc = -(-e // (NT * LCH)) * LCH
    epad = NT * tc

    w1 = W_link[:d_h]
    w2 = W_link[d_h:]
    wdp = conv_W_dp[-1]
    wpd = conv_W_pd[-1]
    def prep(v):
        pad = jnp.full((epad - e,), n, jnp.int32)
        return jnp.concatenate([v.astype(jnp.int32), pad]).reshape(NT, tc)

    idx_all = jnp.stack([prep(edge_index[0]), prep(edge_index[1]),
                         prep(rev_edge_index[0]), prep(rev_edge_index[1])])

    deg_part = _make_deg_kernel(tc)(idx_all)

    blk = 1024
    nb = NACC // blk
    s = pl.pallas_call(
        _mv_kernel,
        grid=(nb,),
        in_specs=[
            pl.BlockSpec((blk, drug_x.shape[1]), lambda i: (i, 0)),
            pl.BlockSpec((blk, protein_x.shape[1]), lambda i: (i, 0)),
            pl.BlockSpec(wdp.shape, lambda i: (0, 0)),
            pl.BlockSpec(w1.shape, lambda i: (0, 0)),
            pl.BlockSpec(wpd.shape, lambda i: (0, 0)),
            pl.BlockSpec(w2.shape, lambda i: (0, 0)),
        ],
        out_specs=pl.BlockSpec((2, blk), lambda i: (0, i)),
        out_shape=jax.ShapeDtypeStruct((2, NACC), jnp.float32),
    )(drug_x, protein_x, wdp, w1, wpd, w2)

    q = pl.pallas_call(
        _q_kernel,
        grid=(nb,),
        in_specs=[
            pl.BlockSpec((2, blk), lambda i: (0, i)),
            pl.BlockSpec((NT, 4, blk), lambda i: (0, 0, i)),
        ],
        out_specs=pl.BlockSpec((4, blk), lambda i: (0, i)),
        out_shape=jax.ShapeDtypeStruct((4, NACC), jnp.float32),
    )(s, deg_part)

    t_part = _make_edge_kernel(tc)(q[:2], idx_all)

    out_full = pl.pallas_call(
        _fin_kernel,
        out_shape=jax.ShapeDtypeStruct((NACC,), jnp.float32),
    )(t_part, q,
      conv_b_dp[-1].reshape(2, d_h // 2), conv_b_pd[-1].reshape(2, d_h // 2),
      w1.reshape(2, d_h // 2), w2.reshape(2, d_h // 2),
      b_link.reshape(1, 1))

    return out_full[:n].reshape(n, 1)
